# baseline (speedup 1.0000x reference)
import functools

import jax
import jax.numpy as jnp
from jax import lax
from jax.experimental import pallas as pl
from jax.experimental.pallas import tpu as pltpu
from jax.experimental.pallas import tpu_sc as plsc

_B, _N, _S, _K = 16, 4096, 1024, 16
_CIN, _COUT = 64, 128
_EPS = 1e-5
_R = _B * _S * _K

_SBLK = 256
_GBLK = 128
_BIGI = 1 << 30


def _fps_body(x_ref, y_ref, z_ref, sx_ref, sy_ref, sz_ref, dmin_ref):
    x = x_ref[...].reshape(_B, _N)
    y = y_ref[...].reshape(_B, _N)
    z = z_ref[...].reshape(_B, _N)
    lane = lax.broadcasted_iota(jnp.int32, (_B, _N), 1)
    scol = lax.broadcasted_iota(jnp.int32, (_B, _S), 1)
    dmin_ref[...] = jnp.full((_B, _N), 1e10, jnp.float32)

    def step(i, carry):
        f, sx, sy, sz = carry
        hit = lane == f
        cx = jnp.sum(jnp.where(hit, x, 0.0), axis=1, keepdims=True)
        cy = jnp.sum(jnp.where(hit, y, 0.0), axis=1, keepdims=True)
        cz = jnp.sum(jnp.where(hit, z, 0.0), axis=1, keepdims=True)
        at_i = scol == i
        sx = jnp.where(at_i, cx, sx)
        sy = jnp.where(at_i, cy, sy)
        sz = jnp.where(at_i, cz, sz)
        py = jnp.minimum((y - cy) * (y - cy), 1e37)
        t1 = (z - cz) * (z - cz) + py
        d = t1 + (x - cx) * (x - cx)
        dm = jnp.minimum(dmin_ref[...], d)
        dmin_ref[...] = dm
        m = jnp.max(dm, axis=1, keepdims=True)
        f = jnp.min(jnp.where(dm == m, lane, _BIGI), axis=1, keepdims=True)
        return f, sx, sy, sz

    f0 = jnp.zeros((_B, 1), jnp.int32)
    z0 = jnp.zeros((_B, _S), jnp.float32)
    _, sx, sy, sz = lax.fori_loop(0, _S, step, (f0, z0, z0, z0))
    sx_ref[...] = sx
    sy_ref[...] = sy
    sz_ref[...] = sz


def _fps(x, y, z):
    return pl.pallas_call(
        _fps_body,
        out_shape=[jax.ShapeDtypeStruct((_B, _S), jnp.float32)] * 3,
        scratch_shapes=[pltpu.VMEM((_B, _N), jnp.float32)],
    )(x, y, z)


def _knn_body(x_ref, y_ref, z_ref, nx_ref, out_ref, d_ref):
    b = pl.program_id(0)
    x = x_ref[0]
    y = y_ref[0]
    z = z_ref[0]
    sel = nx_ref[0]
    sx = sel[:, 0:1]
    sy = sel[:, 1:2]
    sz = sel[:, 2:3]
    xn = x * x + y * y + z * z
    sn = sx * sx + sy * sy + sz * sz
    bf = lambda v: v.astype(jnp.bfloat16).astype(jnp.float32)
    e = bf(sx) * bf(x) + bf(sy) * bf(y) + bf(sz) * bf(z)
    d_ref[...] = (-2.0 * e + sn) + xn
    lane = lax.broadcasted_iota(jnp.int32, (_SBLK, _N), 1)
    off = b * _N
    for k in range(_K):
        d = d_ref[...]
        m = jnp.min(d, axis=1, keepdims=True)
        idx = jnp.min(jnp.where(d == m, lane, _BIGI), axis=1, keepdims=True)
        out_ref[:, :, k : k + 1] = (idx + off)[None]
        d_ref[...] = jnp.where(lane == idx, 1e30, d)


def _knn(x, y, z, new_xyz):
    grid = (_B, _S // _SBLK)
    return pl.pallas_call(
        _knn_body,
        grid=grid,
        in_specs=[
            pl.BlockSpec((1, 1, _N), lambda b, s: (b, 0, 0)),
            pl.BlockSpec((1, 1, _N), lambda b, s: (b, 0, 0)),
            pl.BlockSpec((1, 1, _N), lambda b, s: (b, 0, 0)),
            pl.BlockSpec((1, _SBLK, 3), lambda b, s: (b, s, 0)),
        ],
        out_specs=pl.BlockSpec((1, _SBLK, _K), lambda b, s: (b, s, 0)),
        out_shape=jax.ShapeDtypeStruct((_B, _S, _K), jnp.int32),
        scratch_shapes=[pltpu.VMEM((_SBLK, _N), jnp.float32)],
    )(x, y, z, new_xyz)


def _gather_rows(idx, tbl):
    n_workers = 32
    per_w = _R // n_workers
    ch = 128
    n_ch = per_w // ch
    mesh = plsc.VectorSubcoreMesh(core_axis_name="c", subcore_axis_name="s")

    @functools.partial(
        pl.kernel,
        out_type=jax.ShapeDtypeStruct((_R, _CIN), jnp.float32),
        mesh=mesh,
        scratch_types=[
            pltpu.VMEM((per_w,), jnp.int32),
            pltpu.VMEM((ch, _CIN), jnp.float32),
            pltpu.VMEM((ch, _CIN), jnp.float32),
            pltpu.SemaphoreType.DMA,
            pltpu.SemaphoreType.DMA,
        ],
        compiler_params=pltpu.CompilerParams(use_tc_tiling_on_sc=False),
    )
    def gat(idx_hbm, tbl_hbm, out_hbm, idx_v, rows_a, rows_b, sem_a, sem_b):
        wid = lax.axis_index("s") * 2 + lax.axis_index("c")
        base0 = wid * per_w
        pltpu.sync_copy(idx_hbm.at[pl.ds(base0, per_w)], idx_v)

        def gather(c, buf, sem):
            return pltpu.async_copy(
                tbl_hbm.at[idx_v.at[pl.ds(c * ch, ch)]], buf, sem)

        gather(0, rows_a, sem_a)

        def pair(i, carry):
            c = 2 * i
            gather(c + 1, rows_b, sem_b)
            pltpu.make_async_copy(
                tbl_hbm.at[idx_v.at[pl.ds(c * ch, ch)]], rows_a, sem_a).wait()
            pltpu.sync_copy(rows_a, out_hbm.at[pl.ds(base0 + c * ch, ch)])

            @pl.when(i < n_ch // 2 - 1)
            def _():
                gather(c + 2, rows_a, sem_a)

            pltpu.make_async_copy(
                tbl_hbm.at[idx_v.at[pl.ds((c + 1) * ch, ch)]], rows_b,
                sem_b).wait()
            pltpu.sync_copy(rows_b, out_hbm.at[pl.ds(base0 + (c + 1) * ch, ch)])
            return carry

        lax.fori_loop(0, n_ch // 2, pair, 0)

    return gat(idx, tbl)


def _layer1(g_ref, nx_ref, w1_ref, b1_ref):
    g = g_ref[...].reshape(_GBLK * _K, _CIN)
    h = lax.dot_general(
        g, w1_ref[...], (((1,), (0,)), ((), ())),
        preferred_element_type=jnp.float32, precision=lax.Precision.HIGHEST)
    h = h + b1_ref[...]
    corr = lax.dot_general(
        nx_ref[...], w1_ref[0:3, :], (((1,), (0,)), ((), ())),
        preferred_element_type=jnp.float32, precision=lax.Precision.HIGHEST)
    h3 = h.reshape(_GBLK, _K, _COUT) - corr[:, None, :]
    return h3.reshape(_GBLK * _K, _COUT)


def _acc_stats(p, x, sum_ref, ssq_ref):
    s = jnp.sum(x, axis=0, keepdims=True)
    q = jnp.sum(x * x, axis=0, keepdims=True)

    @pl.when(p == 0)
    def _():
        sum_ref[...] = s
        ssq_ref[...] = q

    @pl.when(p != 0)
    def _():
        sum_ref[...] += s
        ssq_ref[...] += q


def _mlp1_body(g_ref, nx_ref, w1_ref, b1_ref, sum_ref, ssq_ref):
    h1 = _layer1(g_ref, nx_ref, w1_ref, b1_ref)
    _acc_stats(pl.program_id(0), h1, sum_ref, ssq_ref)


def _mlp1(g, nxyz, W1, b1):
    nstep = (_B * _S) // _GBLK
    vec = pl.BlockSpec((1, _COUT), lambda p: (0, 0))
    return pl.pallas_call(
        _mlp1_body,
        grid=(nstep,),
        in_specs=[
            pl.BlockSpec((_GBLK, _K, _CIN), lambda p: (p, 0, 0)),
            pl.BlockSpec((_GBLK, 3), lambda p: (p, 0)),
            pl.BlockSpec((_CIN, _COUT), lambda p: (0, 0)),
            vec,
        ],
        out_specs=[vec, vec],
        out_shape=[
            jax.ShapeDtypeStruct((1, _COUT), jnp.float32),
            jax.ShapeDtypeStruct((1, _COUT), jnp.float32),
        ],
    )(g, nxyz, W1, b1)


def _bn_coeffs(sum_ref, ssq_ref, gamma_ref, beta_ref):
    rn = jnp.float32(1.0 / _R)
    mean = sum_ref[...] * rn
    var = ssq_ref[...] * rn - mean * mean
    scale = gamma_ref[...] * lax.rsqrt(var + _EPS)
    shift = beta_ref[...] - mean * scale
    return scale, shift


def _mlp2_body(g_ref, nx_ref, w1_ref, b1_ref, s1_ref, q1_ref, w2_ref, b2_ref,
               g1_ref, be1_ref, h2_ref, sum_ref, ssq_ref):
    p = pl.program_id(0)
    scale, shift = _bn_coeffs(s1_ref, q1_ref, g1_ref, be1_ref)
    h1 = _layer1(g_ref, nx_ref, w1_ref, b1_ref)
    r = jnp.maximum(h1 * scale + shift, 0.0)
    h2 = lax.dot_general(
        r, w2_ref[...], (((1,), (0,)), ((), ())),
        preferred_element_type=jnp.float32, precision=lax.Precision.HIGHEST)
    h2 = h2 + b2_ref[...]
    h2_ref[...] = h2.reshape(_GBLK, _K, _COUT).astype(jnp.bfloat16)
    _acc_stats(p, h2, sum_ref, ssq_ref)


def _mlp2(g, nxyz, W1, b1, s1, q1, W2, b2, gamma1, beta1):
    nstep = (_B * _S) // _GBLK
    vec = pl.BlockSpec((1, _COUT), lambda p: (0, 0))
    return pl.pallas_call(
        _mlp2_body,
        grid=(nstep,),
        in_specs=[
            pl.BlockSpec((_GBLK, _K, _CIN), lambda p: (p, 0, 0)),
            pl.BlockSpec((_GBLK, 3), lambda p: (p, 0)),
            pl.BlockSpec((_CIN, _COUT), lambda p: (0, 0)),
            vec, vec, vec,
            pl.BlockSpec((_COUT, _COUT), lambda p: (0, 0)),
            vec, vec, vec,
        ],
        out_specs=[
            pl.BlockSpec((_GBLK, _K, _COUT), lambda p: (p, 0, 0)),
            vec, vec,
        ],
        out_shape=[
            jax.ShapeDtypeStruct((_B * _S, _K, _COUT), jnp.bfloat16),
            jax.ShapeDtypeStruct((1, _COUT), jnp.float32),
            jax.ShapeDtypeStruct((1, _COUT), jnp.float32),
        ],
    )(g, nxyz, W1, b1, s1, q1, W2, b2, gamma1, beta1)


def _mlp3_body(h2_ref, s2_ref, q2_ref, g2_ref, be2_ref, out_ref):
    scale, shift = _bn_coeffs(s2_ref, q2_ref, g2_ref, be2_ref)
    h2 = h2_ref[...].astype(jnp.float32)
    r = jnp.maximum(h2 * scale[None] + shift[None], 0.0)
    out_ref[...] = jnp.max(r, axis=1)


def _mlp3(h2, s2, q2, gamma2, beta2):
    nstep = (_B * _S) // _GBLK
    vec = pl.BlockSpec((1, _COUT), lambda p: (0, 0))
    return pl.pallas_call(
        _mlp3_body,
        grid=(nstep,),
        in_specs=[
            pl.BlockSpec((_GBLK, _K, _COUT), lambda p: (p, 0, 0)),
            vec, vec, vec, vec,
        ],
        out_specs=pl.BlockSpec((_GBLK, _COUT), lambda p: (p, 0)),
        out_shape=jax.ShapeDtypeStruct((_B * _S, _COUT), jnp.float32),
    )(h2, s2, q2, gamma2, beta2)


def kernel(xyz, points, W1, b1, gamma1, beta1, W2, b2, gamma2, beta2):
    x = xyz[:, :, 0].reshape(_B, 1, _N)
    y = xyz[:, :, 1].reshape(_B, 1, _N)
    z = xyz[:, :, 2].reshape(_B, 1, _N)

    sx, sy, sz = _fps(x, y, z)
    new_xyz = jnp.stack([sx, sy, sz], axis=-1)

    knn = _knn(x, y, z, new_xyz)

    tbl = jnp.concatenate([xyz, points], axis=-1).reshape(_B * _N, _CIN)
    g = _gather_rows(knn.reshape(_R), tbl).reshape(_B * _S, _K, _CIN)

    nxyz2 = new_xyz.reshape(_B * _S, 3)
    s1, q1 = _mlp1(g, nxyz2, W1, b1[None, :])
    h2, s2, q2 = _mlp2(g, nxyz2, W1, b1[None, :], s1, q1, W2, b2[None, :],
                       gamma1[None, :], beta1[None, :])
    out = _mlp3(h2, s2, q2, gamma2[None, :], beta2[None, :])
    return new_xyz, out.reshape(_B, _S, _COUT)


# speedup vs baseline: 1.0883x; 1.0883x over previous
import functools

import jax
import jax.numpy as jnp
from jax import lax
from jax.experimental import pallas as pl
from jax.experimental.pallas import tpu as pltpu
from jax.experimental.pallas import tpu_sc as plsc

_B, _N, _S, _K = 16, 4096, 1024, 16
_CIN, _COUT = 64, 128
_EPS = 1e-5
_R = _B * _S * _K

_SBLK = 256
_GBLK = 128
_BIGI = 1 << 30


def _fps_body(x_ref, y_ref, z_ref, sx_ref, sy_ref, sz_ref, dmin_ref):
    x = x_ref[...].reshape(_B, _N)
    y = y_ref[...].reshape(_B, _N)
    z = z_ref[...].reshape(_B, _N)
    lane = lax.broadcasted_iota(jnp.int32, (_B, _N), 1)
    scol = lax.broadcasted_iota(jnp.int32, (_B, _S), 1)
    dmin_ref[...] = jnp.full((_B, _N), 1e10, jnp.float32)

    def step(i, carry):
        f, sx, sy, sz = carry
        hit = lane == f
        cx = jnp.sum(jnp.where(hit, x, 0.0), axis=1, keepdims=True)
        cy = jnp.sum(jnp.where(hit, y, 0.0), axis=1, keepdims=True)
        cz = jnp.sum(jnp.where(hit, z, 0.0), axis=1, keepdims=True)
        at_i = scol == i
        sx = jnp.where(at_i, cx, sx)
        sy = jnp.where(at_i, cy, sy)
        sz = jnp.where(at_i, cz, sz)
        py = jnp.minimum((y - cy) * (y - cy), 1e37)
        t1 = (z - cz) * (z - cz) + py
        d = t1 + (x - cx) * (x - cx)
        dm = jnp.minimum(dmin_ref[...], d)
        dmin_ref[...] = dm
        m = jnp.max(dm, axis=1, keepdims=True)
        f = jnp.min(jnp.where(dm == m, lane, _BIGI), axis=1, keepdims=True)
        return f, sx, sy, sz

    f0 = jnp.zeros((_B, 1), jnp.int32)
    z0 = jnp.zeros((_B, _S), jnp.float32)
    _, sx, sy, sz = lax.fori_loop(0, _S, step, (f0, z0, z0, z0))
    sx_ref[...] = sx
    sy_ref[...] = sy
    sz_ref[...] = sz


def _fps(x, y, z):
    return pl.pallas_call(
        _fps_body,
        out_shape=[jax.ShapeDtypeStruct((_B, _S), jnp.float32)] * 3,
        scratch_shapes=[pltpu.VMEM((_B, _N), jnp.float32)],
    )(x, y, z)


def _knn_body(x_ref, y_ref, z_ref, nx_ref, out_ref, d_ref):
    b = pl.program_id(0)
    x = x_ref[0]
    y = y_ref[0]
    z = z_ref[0]
    sel = nx_ref[0]
    sx = sel[:, 0:1]
    sy = sel[:, 1:2]
    sz = sel[:, 2:3]
    xn = x * x + y * y + z * z
    sn = sx * sx + sy * sy + sz * sz
    bf = lambda v: v.astype(jnp.bfloat16).astype(jnp.float32)
    e = bf(sx) * bf(x) + bf(sy) * bf(y) + bf(sz) * bf(z)
    d_ref[...] = (-2.0 * e + sn) + xn
    lane = lax.broadcasted_iota(jnp.int32, (_SBLK, _N), 1)
    off = b * _N
    for k in range(_K):
        d = d_ref[...]
        m = jnp.min(d, axis=1, keepdims=True)
        idx = jnp.min(jnp.where(d == m, lane, _BIGI), axis=1, keepdims=True)
        out_ref[:, :, k : k + 1] = (idx + off)[None]
        d_ref[...] = jnp.where(lane == idx, 1e30, d)


def _knn(x, y, z, new_xyz):
    grid = (_B, _S // _SBLK)
    return pl.pallas_call(
        _knn_body,
        grid=grid,
        in_specs=[
            pl.BlockSpec((1, 1, _N), lambda b, s: (b, 0, 0)),
            pl.BlockSpec((1, 1, _N), lambda b, s: (b, 0, 0)),
            pl.BlockSpec((1, 1, _N), lambda b, s: (b, 0, 0)),
            pl.BlockSpec((1, _SBLK, 3), lambda b, s: (b, s, 0)),
        ],
        out_specs=pl.BlockSpec((1, _SBLK, _K), lambda b, s: (b, s, 0)),
        out_shape=jax.ShapeDtypeStruct((_B, _S, _K), jnp.int32),
        scratch_shapes=[pltpu.VMEM((_SBLK, _N), jnp.float32)],
    )(x, y, z, new_xyz)


def _gather_rows(idx, tbl):
    n_workers = 32
    per_w = _R // n_workers
    ch = 128
    n_ch = per_w // ch
    mesh = plsc.VectorSubcoreMesh(core_axis_name="c", subcore_axis_name="s")

    @functools.partial(
        pl.kernel,
        out_type=jax.ShapeDtypeStruct((_R, _CIN), jnp.float32),
        mesh=mesh,
        scratch_types=[
            pltpu.VMEM((per_w,), jnp.int32),
            pltpu.VMEM((ch, _CIN), jnp.float32),
            pltpu.VMEM((ch, _CIN), jnp.float32),
            pltpu.SemaphoreType.DMA,
            pltpu.SemaphoreType.DMA,
        ],
        compiler_params=pltpu.CompilerParams(use_tc_tiling_on_sc=False),
    )
    def gat(idx_hbm, tbl_hbm, out_hbm, idx_v, rows_a, rows_b, sem_a, sem_b):
        wid = lax.axis_index("s") * 2 + lax.axis_index("c")
        base0 = wid * per_w
        pltpu.sync_copy(idx_hbm.at[pl.ds(base0, per_w)], idx_v)

        def gather(c, buf, sem):
            return pltpu.async_copy(
                tbl_hbm.at[idx_v.at[pl.ds(c * ch, ch)]], buf, sem)

        gather(0, rows_a, sem_a)

        def pair(i, carry):
            c = 2 * i
            gather(c + 1, rows_b, sem_b)
            pltpu.make_async_copy(
                tbl_hbm.at[idx_v.at[pl.ds(c * ch, ch)]], rows_a, sem_a).wait()
            pltpu.sync_copy(rows_a, out_hbm.at[pl.ds(base0 + c * ch, ch)])

            @pl.when(i < n_ch // 2 - 1)
            def _():
                gather(c + 2, rows_a, sem_a)

            pltpu.make_async_copy(
                tbl_hbm.at[idx_v.at[pl.ds((c + 1) * ch, ch)]], rows_b,
                sem_b).wait()
            pltpu.sync_copy(rows_b, out_hbm.at[pl.ds(base0 + (c + 1) * ch, ch)])
            return carry

        lax.fori_loop(0, n_ch // 2, pair, 0)

    return gat(idx, tbl)


def _layer1(g_ref, nx_ref, w1_ref, b1_ref):
    g = g_ref[...].reshape(_GBLK * _K, _CIN)
    h = lax.dot_general(
        g, w1_ref[...], (((1,), (0,)), ((), ())),
        preferred_element_type=jnp.float32, precision=lax.Precision.HIGHEST)
    h = h + b1_ref[...]
    corr = lax.dot_general(
        nx_ref[...], w1_ref[0:3, :], (((1,), (0,)), ((), ())),
        preferred_element_type=jnp.float32, precision=lax.Precision.HIGHEST)
    h3 = h.reshape(_GBLK, _K, _COUT) - corr[:, None, :]
    return h3.reshape(_GBLK * _K, _COUT)


def _acc_stats(p, x, sum_ref, ssq_ref):
    s = jnp.sum(x, axis=0, keepdims=True)
    q = jnp.sum(x * x, axis=0, keepdims=True)

    @pl.when(p == 0)
    def _():
        sum_ref[...] = s
        ssq_ref[...] = q

    @pl.when(p != 0)
    def _():
        sum_ref[...] += s
        ssq_ref[...] += q


def _mlp1_body(g_ref, nx_ref, w1_ref, b1_ref, h1_ref, sum_ref, ssq_ref):
    h1 = _layer1(g_ref, nx_ref, w1_ref, b1_ref)
    h1_ref[...] = h1.reshape(_GBLK, _K, _COUT)
    _acc_stats(pl.program_id(0), h1, sum_ref, ssq_ref)


def _mlp1(g, nxyz, W1, b1):
    nstep = (_B * _S) // _GBLK
    vec = pl.BlockSpec((1, _COUT), lambda p: (0, 0))
    return pl.pallas_call(
        _mlp1_body,
        grid=(nstep,),
        in_specs=[
            pl.BlockSpec((_GBLK, _K, _CIN), lambda p: (p, 0, 0)),
            pl.BlockSpec((_GBLK, 3), lambda p: (p, 0)),
            pl.BlockSpec((_CIN, _COUT), lambda p: (0, 0)),
            vec,
        ],
        out_specs=[
            pl.BlockSpec((_GBLK, _K, _COUT), lambda p: (p, 0, 0)),
            vec, vec,
        ],
        out_shape=[
            jax.ShapeDtypeStruct((_B * _S, _K, _COUT), jnp.float32),
            jax.ShapeDtypeStruct((1, _COUT), jnp.float32),
            jax.ShapeDtypeStruct((1, _COUT), jnp.float32),
        ],
    )(g, nxyz, W1, b1)


def _bn_coeffs(sum_ref, ssq_ref, gamma_ref, beta_ref):
    rn = jnp.float32(1.0 / _R)
    mean = sum_ref[...] * rn
    var = ssq_ref[...] * rn - mean * mean
    scale = gamma_ref[...] * lax.rsqrt(var + _EPS)
    shift = beta_ref[...] - mean * scale
    return scale, shift


def _mlp2_body(h1_ref, s1_ref, q1_ref, w2_ref, b2_ref, g1_ref, be1_ref,
               h2_ref, sum_ref, ssq_ref):
    p = pl.program_id(0)
    scale, shift = _bn_coeffs(s1_ref, q1_ref, g1_ref, be1_ref)
    h1 = h1_ref[...].reshape(_GBLK * _K, _COUT)
    r = jnp.maximum(h1 * scale + shift, 0.0)
    h2 = lax.dot_general(
        r, w2_ref[...], (((1,), (0,)), ((), ())),
        preferred_element_type=jnp.float32, precision=lax.Precision.HIGHEST)
    h2 = h2 + b2_ref[...]
    h2_ref[...] = h2.reshape(_GBLK, _K, _COUT)
    _acc_stats(p, h2, sum_ref, ssq_ref)


def _mlp2(h1, s1, q1, W2, b2, gamma1, beta1):
    nstep = (_B * _S) // _GBLK
    vec = pl.BlockSpec((1, _COUT), lambda p: (0, 0))
    return pl.pallas_call(
        _mlp2_body,
        grid=(nstep,),
        in_specs=[
            pl.BlockSpec((_GBLK, _K, _COUT), lambda p: (p, 0, 0)),
            vec, vec,
            pl.BlockSpec((_COUT, _COUT), lambda p: (0, 0)),
            vec, vec, vec,
        ],
        out_specs=[
            pl.BlockSpec((_GBLK, _K, _COUT), lambda p: (p, 0, 0)),
            vec, vec,
        ],
        out_shape=[
            jax.ShapeDtypeStruct((_B * _S, _K, _COUT), jnp.float32),
            jax.ShapeDtypeStruct((1, _COUT), jnp.float32),
            jax.ShapeDtypeStruct((1, _COUT), jnp.float32),
        ],
    )(h1, s1, q1, W2, b2, gamma1, beta1)


def _mlp3_body(h2_ref, s2_ref, q2_ref, g2_ref, be2_ref, out_ref):
    scale, shift = _bn_coeffs(s2_ref, q2_ref, g2_ref, be2_ref)
    h2 = h2_ref[...]
    r = jnp.maximum(h2 * scale[None] + shift[None], 0.0)
    out_ref[...] = jnp.max(r, axis=1)


def _mlp3(h2, s2, q2, gamma2, beta2):
    nstep = (_B * _S) // _GBLK
    vec = pl.BlockSpec((1, _COUT), lambda p: (0, 0))
    return pl.pallas_call(
        _mlp3_body,
        grid=(nstep,),
        in_specs=[
            pl.BlockSpec((_GBLK, _K, _COUT), lambda p: (p, 0, 0)),
            vec, vec, vec, vec,
        ],
        out_specs=pl.BlockSpec((_GBLK, _COUT), lambda p: (p, 0)),
        out_shape=jax.ShapeDtypeStruct((_B * _S, _COUT), jnp.float32),
    )(h2, s2, q2, gamma2, beta2)


def kernel(xyz, points, W1, b1, gamma1, beta1, W2, b2, gamma2, beta2):
    x = xyz[:, :, 0].reshape(_B, 1, _N)
    y = xyz[:, :, 1].reshape(_B, 1, _N)
    z = xyz[:, :, 2].reshape(_B, 1, _N)

    sx, sy, sz = _fps(x, y, z)
    new_xyz = jnp.stack([sx, sy, sz], axis=-1)

    knn = _knn(x, y, z, new_xyz)

    tbl = jnp.concatenate([xyz, points], axis=-1).reshape(_B * _N, _CIN)
    g = _gather_rows(knn.reshape(_R), tbl).reshape(_B * _S, _K, _CIN)

    nxyz2 = new_xyz.reshape(_B * _S, 3)
    h1, s1, q1 = _mlp1(g, nxyz2, W1, b1[None, :])
    h2, s2, q2 = _mlp2(h1, s1, q1, W2, b2[None, :], gamma1[None, :],
                       beta1[None, :])
    out = _mlp3(h2, s2, q2, gamma2[None, :], beta2[None, :])
    return new_xyz, out.reshape(_B, _S, _COUT)
